# baseline (speedup 1.0000x reference)
import functools

import jax
import jax.numpy as jnp
from jax.experimental import pallas as pl


def _leaky(x):
    return jnp.where(x > 0, x, 0.01 * x)


def _edge_mlp_block(pre_ref, wb_ref, bb_ref, out_ref):
    t = _leaky(pre_ref[...])
    out_ref[...] = jnp.dot(t, wb_ref[...], preferred_element_type=jnp.float32) + bb_ref[...]


def _edge_mlp(pre, Wb, bb, block_e=8192):
    ne = pre.shape[0]
    grid = (ne // block_e,)
    return pl.pallas_call(
        _edge_mlp_block,
        grid=grid,
        in_specs=[
            pl.BlockSpec((block_e, 8), lambda i: (i, 0)),
            pl.BlockSpec((8, 8), lambda i: (0, 0)),
            pl.BlockSpec((1, 8), lambda i: (0, 0)),
        ],
        out_specs=pl.BlockSpec((block_e, 8), lambda i: (i, 0)),
        out_shape=jax.ShapeDtypeStruct((ne, 8), jnp.float32),
    )(pre, Wb, bb.reshape(1, 8))


def kernel(node_features, edge_index, edge_features, actions,
           W1a, b1a, W1b, b1b, W2a, b2a, W2b, b2b, W3a, b3a, W3b, b3b, Wl, bl):
    Bn, Nn = node_features.shape[0], node_features.shape[1]
    x = jnp.concatenate([node_features.astype(jnp.float32), actions[..., None]], axis=-1)
    x = x.reshape(Bn * Nn, -1)
    offs = (jnp.arange(Bn, dtype=edge_index.dtype) * Nn)[:, None, None]
    ei = (edge_index + offs).transpose(1, 0, 2).reshape(2, -1)
    ea = edge_features.reshape(-1)
    src, dst = ei[0], ei[1]
    num = Bn * Nn

    def conv(h, Wa, ba, Wb, bb):
        F = h.shape[1]
        A = h @ Wa[:F]
        Bm = h @ Wa[F:2 * F]
        wa_e = Wa[2 * F]
        pre = A[dst] + Bm[src] + ea[:, None] * wa_e[None, :] + ba[None, :]
        m = _edge_mlp(pre, Wb, bb)
        agg = jax.ops.segment_min(m, dst, num_segments=num)
        return jnp.where(jnp.isfinite(agg), agg, 0.0)

    h1 = _leaky(conv(x, W1a, b1a, W1b, b1b))
    h2 = _leaky(conv(h1, W2a, b2a, W2b, b2b))
    h3 = _leaky(conv(h2, W3a, b3a, W3b, b3b))
    xc = jnp.concatenate([x, h3], axis=1)
    xs = xc.reshape(Bn, Nn, -1).sum(axis=1)
    return xs @ Wl + bl


# speedup vs baseline: 6.5405x; 6.5405x over previous
import functools

import jax
import jax.numpy as jnp
from jax import lax
from jax.experimental import pallas as pl
from jax.experimental.pallas import tpu as pltpu
from jax.experimental.pallas import tpu_sc as plsc

NUM = 100000
TE = 3200000
NW = 32
NPW = NUM // NW
EPW = 102400
EPAD = EPW * NW
W3 = 12800
NWIN3 = EPW // W3
W4 = 2048
CAP = EPAD + 3072
NB = 33
HN = NB * 16
INF = float("inf")
_DIAG_XLA_LAYERS = False
DIV_M = 21475
DIV_S = 26

_MESH = plsc.VectorSubcoreMesh(core_axis_name="c", subcore_axis_name="s",
                               num_cores=2, num_subcores=16)
_SC_PARAMS = pltpu.CompilerParams(needs_layout_passes=False,
                                  use_tc_tiling_on_sc=False)


def _wid():
    return lax.axis_index("s") * 2 + lax.axis_index("c")


def _lane():
    return lax.iota(jnp.int32, 16)


def _scalar(vec, lane):
    return jnp.sum(jnp.where(_lane() == lane, vec, 0), axis=0)


def _splat(x):
    return lax.broadcast_in_dim(jnp.asarray(x, jnp.int32), (16,), ())


def _bucket(d):
    du = d.astype(jnp.uint32)
    return ((du * jnp.uint32(DIV_M)) >> DIV_S).astype(jnp.int32)


_GD = lax.GatherDimensionNumbers(offset_dims=(), collapsed_slice_dims=(0,),
                                 start_index_map=(0,))


def _vperm(x, idx):
    return lax.gather(x, idx[:, None], _GD, (1,),
                      mode=lax.GatherScatterMode.PROMISE_IN_BOUNDS)


def _count_body(dst_hbm, cnt_hbm, hist, dstc):
    wid = _wid()
    lanes = _lane()
    zero16 = jnp.zeros((16,), jnp.int32)
    ones = jnp.ones((16,), jnp.int32)

    def zb(i, _):
        hist[pl.ds(i * 16, 16)] = zero16
        return 0

    lax.fori_loop(0, NB, zb, 0)
    for win in range(NWIN3):
        g0 = wid * EPW + win * W3
        pltpu.sync_copy(dst_hbm.at[pl.ds(g0, W3)], dstc)

        def body(i, _):
            d = dstc[pl.ds(i * 16, 16)]
            idx = _bucket(d) * 16 + lanes
            plsc.addupdate_scatter(hist, [idx], ones)
            return 0

        lax.fori_loop(0, W3 // 16, body, 0)
    pltpu.sync_copy(hist, cnt_hbm.at[pl.ds(wid * HN, HN)])


def _count_edges(dst1d):
    return pl.kernel(
        _count_body,
        out_type=jax.ShapeDtypeStruct((NW * HN,), jnp.int32),
        mesh=_MESH,
        compiler_params=_SC_PARAMS,
        scratch_types=[
            pltpu.VMEM((HN,), jnp.int32),
            pltpu.VMEM((W3,), jnp.int32),
        ],
    )(dst1d)


def _scatter_body(cnt_hbm, dst_hbm, src_hbm, ea_hbm,
                  sdst_hbm, ssrc_hbm, sea_hbm, bounds_hbm,
                  cloc, ow, dstc, srcc, eac, slotb, sem):
    wid = _wid()
    lanes = _lane()
    pltpu.sync_copy(cnt_hbm, cloc)

    def scan_body(j, carry):
        tot, boff, end = carry
        b = j // NW
        w = j % NW
        v = cloc[pl.ds(w * HN + b * 16, 16)]
        excl = plsc.cumsum(v) - v + _splat(tot)
        vsum = jnp.sum(v, axis=0)

        @pl.when(w == wid)
        def _():
            ow[pl.ds(b * 16, 16)] = excl

        boff = boff + jnp.where(b < wid, vsum, 0)
        end = end + jnp.where(b <= wid, vsum, 0)
        return tot + vsum, boff, end

    _, boff, end = lax.fori_loop(0, NB * NW, scan_body,
                                 (jnp.int32(0), jnp.int32(0), jnp.int32(0)))
    bvec = jnp.where(lanes == 0, _splat(boff),
                     jnp.where(lanes == 1, _splat(end), 0))
    slotb[0, pl.ds(0, 16)] = bvec
    pltpu.sync_copy(slotb.at[0, pl.ds(0, 16)],
                    bounds_hbm.at[pl.ds(wid * 16, 16)])

    for win in range(NWIN3):
        g0 = wid * EPW + win * W3
        pltpu.sync_copy(dst_hbm.at[pl.ds(g0, W3)], dstc)
        pltpu.sync_copy(src_hbm.at[pl.ds(g0, W3)], srcc)
        pltpu.sync_copy(ea_hbm.at[pl.ds(g0, W3)], eac)

        def body(i, _):
            d = dstc[pl.ds(i * 16, 16)]
            idx = _bucket(d) * 16 + lanes
            slot = plsc.load_gather(ow, [idx])
            plsc.store_scatter(ow, [idx], slot + 1)
            slotb[i // 8, pl.ds((i % 8) * 16, 16)] = slot
            return 0

        lax.fori_loop(0, W3 // 16, body, 0)

        def dma_body(j, _):
            r = pl.ds(j * 128, 128)
            c1 = pltpu.async_copy(dstc.at[r], sdst_hbm.at[slotb.at[j]], sem)
            c2 = pltpu.async_copy(srcc.at[r], ssrc_hbm.at[slotb.at[j]], sem)
            c3 = pltpu.async_copy(eac.at[r], sea_hbm.at[slotb.at[j]], sem)
            c1.wait()
            c2.wait()
            c3.wait()
            return 0

        lax.fori_loop(0, W3 // 128, dma_body, 0)


def _scatter_edges(cnt, dst1d, src1d, ea1d):
    return pl.kernel(
        _scatter_body,
        out_type=(
            jax.ShapeDtypeStruct((CAP,), jnp.int32),
            jax.ShapeDtypeStruct((CAP,), jnp.int32),
            jax.ShapeDtypeStruct((CAP,), jnp.float32),
            jax.ShapeDtypeStruct((NW * 16,), jnp.int32),
        ),
        mesh=_MESH,
        compiler_params=_SC_PARAMS,
        scratch_types=[
            pltpu.VMEM((NW * HN,), jnp.int32),
            pltpu.VMEM((HN,), jnp.int32),
            pltpu.VMEM((W3,), jnp.int32),
            pltpu.VMEM((W3,), jnp.int32),
            pltpu.VMEM((W3,), jnp.float32),
            pltpu.VMEM((W3 // 128, 128), jnp.int32),
            pltpu.SemaphoreType.DMA,
        ],
    )(cnt, dst1d, src1d, ea1d)


def _layer_body(a_hbm, b_hbm, bounds_hbm, sdst_hbm, ssrc_hbm, sea_hbm,
                wb_hbm, wavbb_hbm, agg_hbm,
                aloc, tbl, dstc, srcc, eac, bvrows, w8, wv, sem):
    wid = _wid()
    lanes = _lane()
    half = (lanes >= 8).astype(jnp.int32)
    dim8 = lanes & 7
    swap = (lanes + 8) & 15
    inf16 = jnp.full((16,), INF, jnp.float32)

    pltpu.sync_copy(a_hbm.at[pl.ds(wid * (NPW * 8), NPW * 8)], aloc)
    pltpu.sync_copy(wb_hbm, w8)
    pltpu.sync_copy(wavbb_hbm, wv)
    wav = wv[pl.ds(0, 16)]
    bb2 = wv[pl.ds(16, 16)]
    wbrows = [w8[pl.ds(k * 16, 16)] for k in range(8)]

    def init_body(i, _):
        tbl[pl.ds(i * 16, 16)] = inf16
        return 0

    lax.fori_loop(0, (NPW * 8 + 88) // 16, init_body, 0)

    pltpu.sync_copy(bounds_hbm.at[pl.ds(wid * 16, 16)], srcc.at[pl.ds(0, 16)])
    bvec = srcc[pl.ds(0, 16)]
    boff = _scalar(bvec, 0)
    end = _scalar(bvec, 1)
    start = (boff // 8) * 8
    nwin = (end - start + (W4 - 1)) // W4

    def win_body(win, _):
        g0 = start + win * W4
        pltpu.sync_copy(sdst_hbm.at[pl.ds(g0, W4)], dstc)
        pltpu.sync_copy(ssrc_hbm.at[pl.ds(g0, W4)], srcc)
        pltpu.sync_copy(sea_hbm.at[pl.ds(g0, W4)], eac)

        wbase = _splat(wid * NPW)

        def clamp_body(i, _):
            s = srcc[pl.ds(i * 16, 16)]
            srcc[pl.ds(i * 16, 16)] = jnp.clip(s, 0, NUM - 1)
            d = dstc[pl.ds(i * 16, 16)]
            dstc[pl.ds(i * 16, 16)] = jnp.clip(d - wbase, 0, NPW - 1)
            return 0

        lax.fori_loop(0, W4 // 16, clamp_body, 0)

        cps = [pltpu.async_copy(b_hbm.at[srcc.at[pl.ds(j * 128, 128)]],
                                bvrows.at[pl.ds(j * 128, 128)], sem)
               for j in range(W4 // 128)]
        for cp in cps:
            cp.wait()

        boffv = _splat(boff)
        endv = _splat(end)
        g0v = _splat(g0)

        def body(i, _):
            k2 = _splat(2 * i) + half
            dl = plsc.load_gather(dstc, [k2])
            eav = plsc.load_gather(eac, [k2])
            idxt = dl * 8 + dim8
            av = plsc.load_gather(aloc, [idxt])
            bvv = plsc.load_gather(bvrows, [k2, dim8])
            pre = av + bvv + eav * wav
            t = jnp.where(pre > 0, pre, 0.01 * pre)
            m = bb2
            for kk in range(8):
                tk = _vperm(t, kk + 8 * half)
                m = m + tk * wbrows[kk]
            gs = g0v + k2
            maskv = (gs >= boffv) & (gs < endv)
            mvi = maskv.astype(jnp.int32)
            msw = _vperm(m, swap)
            dsw = _vperm(dl, swap)
            msk_sw = _vperm(mvi, swap)
            m = jnp.where((dl == dsw) & (msk_sw == 1), jnp.minimum(m, msw), m)
            cur = plsc.load_gather(tbl, [idxt])
            plsc.store_scatter(tbl, [idxt], jnp.minimum(cur, m), mask=maskv)
            return 0

        lax.fori_loop(0, W4 // 2, body, 0)
        return 0

    lax.fori_loop(0, nwin, win_body, 0)
    pltpu.sync_copy(tbl.at[pl.ds(0, NPW * 8)],
                    agg_hbm.at[pl.ds(wid * (NPW * 8), NPW * 8)])


def _layer_sc(a2, b2, bounds, sdst, ssrc, sea, wb2, wavbb):
    return pl.kernel(
        _layer_body,
        out_type=jax.ShapeDtypeStruct((NUM * 8,), jnp.float32),
        mesh=_MESH,
        compiler_params=_SC_PARAMS,
        scratch_types=[
            pltpu.VMEM((NPW * 8,), jnp.float32),
            pltpu.VMEM((NPW * 8 + 88,), jnp.float32),
            pltpu.VMEM((W4,), jnp.int32),
            pltpu.VMEM((W4,), jnp.int32),
            pltpu.VMEM((W4,), jnp.float32),
            pltpu.VMEM((W4, 8), jnp.float32),
            pltpu.VMEM((128,), jnp.float32),
            pltpu.VMEM((32,), jnp.float32),
            pltpu.SemaphoreType.DMA,
        ],
    )(a2, b2, bounds, sdst, ssrc, sea, wb2, wavbb)


def _proj_block(post, h_ref, wd_ref, ws_ref, ba_ref, a_ref, b_ref):
    h = h_ref[...]
    if post:
        h = jnp.where(jnp.isfinite(h), h, 0.0)
        h = jnp.where(h > 0, h, 0.01 * h)
    a_ref[...] = jnp.dot(h, wd_ref[...], preferred_element_type=jnp.float32) + ba_ref[...]
    b_ref[...] = jnp.dot(h, ws_ref[...], preferred_element_type=jnp.float32)


def _proj(h, wdst, wsrc, ba, post, blk=10000):
    f = h.shape[1]
    grid = (NUM // blk,)
    return pl.pallas_call(
        functools.partial(_proj_block, post),
        grid=grid,
        in_specs=[
            pl.BlockSpec((blk, f), lambda i: (i, 0)),
            pl.BlockSpec((f, 8), lambda i: (0, 0)),
            pl.BlockSpec((f, 8), lambda i: (0, 0)),
            pl.BlockSpec((1, 8), lambda i: (0, 0)),
        ],
        out_specs=[
            pl.BlockSpec((blk, 8), lambda i: (i, 0)),
            pl.BlockSpec((blk, 8), lambda i: (i, 0)),
        ],
        out_shape=[
            jax.ShapeDtypeStruct((NUM, 8), jnp.float32),
            jax.ShapeDtypeStruct((NUM, 8), jnp.float32),
        ],
    )(h, wdst, wsrc, ba.reshape(1, 8))


def _readout_block(x_ref, agg_ref, wl3_ref, wl8_ref, out_ref):
    h3 = agg_ref[...]
    h3 = jnp.where(jnp.isfinite(h3), h3, 0.0)
    h3 = jnp.where(h3 > 0, h3, 0.01 * h3)
    s3 = jnp.dot(x_ref[...], wl3_ref[...], preferred_element_type=jnp.float32)
    s8 = jnp.dot(h3, wl8_ref[...], preferred_element_type=jnp.float32)
    out_ref[...] = jnp.broadcast_to(jnp.sum(s3 + s8), (8, 128))


def _readout(x, agg3, wl, blk=5000):
    grid = (NUM // blk,)
    part = pl.pallas_call(
        _readout_block,
        grid=grid,
        in_specs=[
            pl.BlockSpec((blk, 3), lambda i: (i, 0)),
            pl.BlockSpec((blk, 8), lambda i: (i, 0)),
            pl.BlockSpec((3, 1), lambda i: (0, 0)),
            pl.BlockSpec((8, 1), lambda i: (0, 0)),
        ],
        out_specs=pl.BlockSpec((8, 128), lambda i: (i, 0)),
        out_shape=jax.ShapeDtypeStruct((NUM // blk * 8, 128), jnp.float32),
    )(x, agg3, wl[:3], wl[3:])
    return part[::8, 0]


def kernel(node_features, edge_index, edge_features, actions,
           W1a, b1a, W1b, b1b, W2a, b2a, W2b, b2b, W3a, b3a, W3b, b3b, Wl, bl):
    Bn, Nn = node_features.shape[0], node_features.shape[1]
    x = jnp.concatenate([node_features.astype(jnp.float32), actions[..., None]],
                        axis=-1).reshape(Bn * Nn, 3)
    offs = (jnp.arange(Bn, dtype=edge_index.dtype) * Nn)[:, None, None]
    ei = (edge_index + offs).transpose(1, 0, 2).reshape(2, -1)
    ea = edge_features.reshape(-1)
    src, dst = ei[0], ei[1]

    pad = EPAD - TE
    dst1d = jnp.concatenate([dst, jnp.full((pad,), NUM, dst.dtype)])
    src1d = jnp.concatenate([src, jnp.zeros((pad,), src.dtype)])
    ea1d = jnp.concatenate([ea, jnp.zeros((pad,), ea.dtype)])

    cnt = _count_edges(dst1d)
    sdst, ssrc, sea, bounds = _scatter_edges(cnt, dst1d, src1d, ea1d)

    if _DIAG_XLA_LAYERS:
        sd = jnp.clip(sdst[:EPAD], 0, NUM)
        ss = jnp.clip(ssrc[:EPAD], 0, NUM - 1)
        se = sea[:EPAD]

        def dconv(h, Wa, ba, Wb, bb):
            f = h.shape[1]
            A = h @ Wa[:f] + ba
            Bm = h @ Wa[f:2 * f]
            pre = A[jnp.minimum(sd, NUM - 1)] + Bm[ss] + \
                se[:, None] * Wa[2 * f][None, :]
            m = jnp.where(pre > 0, pre, 0.01 * pre) @ Wb + bb
            agg = jax.ops.segment_min(m, sd, num_segments=NUM + 1)[:NUM]
            return jnp.where(jnp.isfinite(agg), agg, 0.0)

        def lk(v):
            return jnp.where(v > 0, v, 0.01 * v)

        h1 = lk(dconv(x, W1a, b1a, W1b, b1b))
        h2 = lk(dconv(h1, W2a, b2a, W2b, b2b))
        h3r = dconv(h2, W3a, b3a, W3b, b3b)
        part = _readout(x, h3r, Wl)
        return part.reshape(Bn, -1).sum(axis=1, keepdims=True) + bl

    def conv(h, Wa, ba, Wb, bb, post):
        f = h.shape[1]
        a2, b2 = _proj(h, Wa[:f], Wa[f:2 * f], ba, post)
        wavbb = jnp.concatenate([jnp.tile(Wa[2 * f], 2), jnp.tile(bb, 2)])
        wb2 = jnp.concatenate([Wb, Wb], axis=1).reshape(-1)
        agg = _layer_sc(a2.reshape(-1), b2, bounds, sdst, ssrc, sea, wb2, wavbb)
        return agg.reshape(NUM, 8)

    agg1 = conv(x, W1a, b1a, W1b, b1b, post=False)
    agg2 = conv(agg1, W2a, b2a, W2b, b2b, post=True)
    agg3 = conv(agg2, W3a, b3a, W3b, b3b, post=True)

    part = _readout(x, agg3, Wl)
    out = part.reshape(Bn, -1).sum(axis=1, keepdims=True) + bl
    return out
